# baseline (speedup 1.0000x reference)
import functools

import jax
import jax.numpy as jnp
from jax.experimental import pallas as pl
from jax.experimental.pallas import tpu as pltpu

B_, L_, D_ = 8, 2048, 1024
BD_ = 512
DIN_ = 1024
NST_ = 16
KC_ = 4
DTR_ = 32

G_ = 8
T_ = 128
SB_ = 16
GT_ = G_ * T_


def _silu(v):
    return v / (1.0 + jnp.exp(-v))


def _mamba_kernel(
    x_ref, wd_ref, bdn_ref, wia_ref, wib_ref, cw_ref, cb_ref,
    wx1_ref, wdt_ref, bdt_ref, wbt_ref, wc_ref, an_ref, dsk_ref,
    wo_ref, lng_ref, lnb_ref, wu_ref, bup_ref,
    out_ref,
    xbuf, z_s, xc_s, dt_s, u_s, cc_s, y_s, hstk, h_s,
):
    c = pl.program_id(1)
    bf16 = jnp.bfloat16

    @pl.when(c == 0)
    def _init():
        h_s[...] = jnp.zeros((G_ * NST_, DIN_), jnp.float32)
        for g in range(G_):
            xbuf[g, pl.ds(0, 8), :] = jnp.zeros((8, DIN_), jnp.float32)

    xv = x_ref[...].reshape(GT_, D_)
    hdn = jnp.dot(xv, wd_ref[...], preferred_element_type=jnp.float32)
    hdn = (hdn + bdn_ref[...]).astype(bf16)
    ina = jnp.dot(hdn, wia_ref[...], preferred_element_type=jnp.float32)
    z_s[...] = jnp.dot(hdn, wib_ref[...],
                       preferred_element_type=jnp.float32)
    for g in range(G_):
        xbuf[g, pl.ds(8, T_), :] = ina[g * T_:(g + 1) * T_]

    for g in range(G_):
        xwin = xbuf[g]
        conv = cb_ref[...]
        for k in range(KC_):
            conv = conv + xwin[5 + k:5 + k + T_, :] * cw_ref[k:k + 1, :]
        xc_s[pl.ds(g * T_, T_), :] = _silu(conv)
        xbuf[g, pl.ds(0, 8), :] = xwin[T_:T_ + 8, :]

    xcv = xc_s[...]
    xcb = xcv.astype(bf16)
    dt_in = jnp.dot(xcb, wx1_ref[...],
                    preferred_element_type=jnp.float32).astype(bf16)
    dt_lin = jnp.dot(dt_in, wdt_ref[...],
                     preferred_element_type=jnp.float32) + bdt_ref[...]
    dtv = jnp.maximum(dt_lin, 0.0) + jnp.log(
        1.0 + jnp.exp(-jnp.abs(dt_lin)))
    dt_s[...] = dtv
    u_s[...] = dtv * xcv
    bt_all = jax.lax.dot_general(wbt_ref[...], xcb, (((1,), (1,)), ((), ())),
                                 preferred_element_type=jnp.float32)
    cc_s[...] = jnp.dot(xcb, wc_ref[...],
                        preferred_element_type=jnp.float32)

    bts = [bt_all[:, g * T_:(g + 1) * T_] for g in range(G_)]

    def sub_block(i, _):
        anv = an_ref[...]
        blk = NST_ * SB_
        lane_b = jax.lax.broadcasted_iota(jnp.int32, (NST_, T_), 1)
        lane8 = jax.lax.broadcasted_iota(jnp.int32, (SB_, blk), 1)
        row8 = jax.lax.broadcasted_iota(jnp.int32, (SB_, blk), 0)
        cbds = []
        hs = [h_s[pl.ds(g * NST_, NST_), :] for g in range(G_)]
        for g in range(G_):
            dt8 = dt_s[pl.ds(g * T_ + i * SB_, SB_), :]
            u8 = u_s[pl.ds(g * T_ + i * SB_, SB_), :]
            c8 = cc_s[pl.ds(g * T_ + i * SB_, SB_), :]
            cbds.append(jnp.where((lane8 >> 4) == row8,
                                  jnp.tile(c8, (1, SB_)), 0.0))
            h = hs[g]
            for j in range(SB_):
                t = i * SB_ + j
                bcol = jnp.sum(jnp.where(lane_b == t, bts[g], 0.0), axis=1,
                               keepdims=True)
                dA = jnp.exp2(dt8[j:j + 1, :] * anv)
                h = h * dA + u8[j:j + 1, :] * bcol
                hstk[pl.ds(g * (NST_ * SB_) + NST_ * j, NST_), :] = h.astype(
                    bf16)
            h_s[pl.ds(g * NST_, NST_), :] = h
        gl = G_ * blk
        lsh = blk.bit_length() - 1
        rsh = SB_.bit_length() - 1
        cbd4 = jnp.concatenate(cbds, axis=1)
        rowg = jax.lax.broadcasted_iota(jnp.int32, (G_ * SB_, gl), 0)
        laneg = jax.lax.broadcasted_iota(jnp.int32, (G_ * SB_, gl), 1)
        cbd = jnp.where((laneg >> lsh) == (rowg >> rsh),
                        jnp.tile(cbd4, (G_, 1)), 0.0)
        y32 = jnp.dot(cbd.astype(bf16), hstk[...],
                      preferred_element_type=jnp.float32)
        for g in range(G_):
            y_s[pl.ds(g * T_ + i * SB_, SB_), :] = y32[g * SB_:(g + 1) * SB_]
        return 0

    jax.lax.fori_loop(0, T_ // SB_, sub_block, 0)

    yv = (y_s[...] + dsk_ref[...] * xc_s[...]) * _silu(z_s[...])
    m = jnp.dot(yv.astype(bf16), wo_ref[...],
                preferred_element_type=jnp.float32)
    mu = jnp.mean(m, axis=-1, keepdims=True)
    dmu = m - mu
    var = jnp.mean(dmu * dmu, axis=-1, keepdims=True)
    mn = dmu * jax.lax.rsqrt(var + 1e-5) * lng_ref[...] + lnb_ref[...]
    out = jnp.dot(mn.astype(bf16), wu_ref[...],
                  preferred_element_type=jnp.float32) + bup_ref[...]
    out_ref[...] = out.reshape(G_, T_, D_)


@functools.partial(jax.jit, static_argnames=())
def kernel(x, W_down, b_down, W_in, conv_w, conv_b, W_x, W_dt, b_dt,
           A_log, D_skip, W_out, ln_g, ln_b, W_up, b_up):
    f32 = jnp.float32
    bf16 = jnp.bfloat16
    wia = W_in[:, :DIN_].astype(bf16)
    wib = W_in[:, DIN_:].astype(bf16)
    cw = conv_w.T
    wx1 = W_x[:, :DTR_].astype(bf16)
    wbt = W_x[:, DTR_:DTR_ + NST_].T.astype(bf16)
    wc = W_x[:, DTR_ + NST_:].astype(bf16)
    an = (-jnp.exp(A_log)).T * 1.4426950408889634
    row = lambda v: v.reshape(1, -1).astype(f32)

    nchunks = L_ // T_
    grid = (B_ // G_, nchunks)
    full = lambda shp: pl.BlockSpec(shp, lambda b, c: (0,) * len(shp))
    specs = [
        pl.BlockSpec((G_, T_, D_), lambda b, c: (b, c, 0)),
        full((D_, BD_)),
        full((1, BD_)),
        full((BD_, DIN_)),
        full((BD_, DIN_)),
        full((KC_, DIN_)),
        full((1, DIN_)),
        full((DIN_, DTR_)),
        full((DTR_, DIN_)),
        full((1, DIN_)),
        full((NST_, DIN_)),
        full((DIN_, NST_)),
        full((NST_, DIN_)),
        full((1, DIN_)),
        full((DIN_, BD_)),
        full((1, BD_)),
        full((1, BD_)),
        full((BD_, D_)),
        full((1, D_)),
    ]
    scratch = [
        pltpu.VMEM((G_, T_ + 8, DIN_), f32),
        pltpu.VMEM((GT_, DIN_), f32),
        pltpu.VMEM((GT_, DIN_), f32),
        pltpu.VMEM((GT_, DIN_), f32),
        pltpu.VMEM((GT_, DIN_), f32),
        pltpu.VMEM((GT_, NST_), f32),
        pltpu.VMEM((GT_, DIN_), f32),
        pltpu.VMEM((G_ * NST_ * SB_, DIN_), bf16),
        pltpu.VMEM((G_ * NST_, DIN_), f32),
    ]
    out = pl.pallas_call(
        _mamba_kernel,
        grid=grid,
        in_specs=specs,
        out_specs=pl.BlockSpec((G_, T_, D_), lambda b, c: (b, c, 0)),
        out_shape=jax.ShapeDtypeStruct((B_, L_, D_), f32),
        scratch_shapes=scratch,
        compiler_params=pltpu.CompilerParams(
            dimension_semantics=("parallel", "arbitrary")),
    )(x.astype(bf16), W_down.astype(bf16), row(b_down), wia, wib, cw,
      row(conv_b), wx1, W_dt.astype(bf16), row(b_dt), wbt, wc, an,
      row(D_skip), W_out.astype(bf16), row(ln_g), row(ln_b),
      W_up.astype(bf16), row(b_up))
    return out
